# baseline (speedup 1.0000x reference)
import functools

import jax
import jax.numpy as jnp
from jax import lax
from jax.experimental import pallas as pl
from jax.experimental.pallas import tpu as pltpu
from jax.experimental.pallas import tpu_sc as plsc

N = 10000
D = 128
E = 320000

NC = 2
NS = 16
NW = NC * NS

NPAD = 10240
RPT = NPAD // NS
CHT = 80
EPT = CHT * 128
E_PAD = EPT * NW

BR = 512
GRID = NPAD // BR

_MESH = plsc.VectorSubcoreMesh(
    core_axis_name="c", subcore_axis_name="s", num_cores=NC, num_subcores=NS)


@functools.partial(
    pl.kernel,
    out_type=jax.ShapeDtypeStruct((NW, NPAD), jnp.float32),
    mesh=_MESH,
    compiler_params=pltpu.CompilerParams(needs_layout_passes=False),
    scratch_types=[
        pltpu.VMEM((EPT,), jnp.int32),
        pltpu.VMEM((NPAD,), jnp.float32),
    ],
)
def _deg_kernel(dst_hbm, out_hbm, dst_v, acc_v):
    c = lax.axis_index("c")
    s = lax.axis_index("s")
    w = s * NC + c
    pltpu.sync_copy(dst_hbm.at[pl.ds(w * EPT, EPT)], dst_v)
    zeros = jnp.zeros((16,), jnp.float32)

    def zbody(i, carry):
        acc_v[pl.ds(i * 16, 16)] = zeros
        return carry

    lax.fori_loop(0, NPAD // 16, zbody, 0)
    ones = jnp.ones((16,), jnp.float32)

    def body(i, carry):
        idx = dst_v[pl.ds(i * 16, 16)]
        plsc.addupdate_scatter(acc_v, [idx], ones)
        return carry

    lax.fori_loop(0, EPT // 16, body, 0)
    pltpu.sync_copy(acc_v, out_hbm.at[w])


@functools.partial(
    pl.kernel,
    out_type=jax.ShapeDtypeStruct((NC, NPAD, D), jnp.float32),
    mesh=_MESH,
    scratch_types=[
        pltpu.VMEM((CHT, 128), jnp.int32),
        pltpu.VMEM((2, 128), jnp.int32),
        pltpu.VMEM((2, 128), jnp.int32),
        pltpu.VMEM((128, D), jnp.float32),
        pltpu.VMEM((128, D), jnp.float32),
        pltpu.VMEM_SHARED((NPAD, D), jnp.float32),
        pltpu.SemaphoreType.DMA,
        pltpu.SemaphoreType.DMA,
    ],
)
def _scatter_kernel(y_hbm, pk_hbm, zero_hbm, out_hbm,
                    pk_v, sst, dstg, r0, r1, acc_sh, g0, g1):
    c = lax.axis_index("c")
    s = lax.axis_index("s")
    w = s * NC + c
    rows = (r0, r1)
    gsem = (g0, g1)

    pltpu.sync_copy(zero_hbm.at[pl.ds(s * RPT, RPT)],
                    acc_sh.at[pl.ds(s * RPT, RPT)])

    pltpu.sync_copy(pk_hbm.at[w], pk_v)
    plsc.subcore_barrier()

    def unpack(j, p):
        for k in range(8):
            v = pk_v[j, pl.ds(16 * k, 16)]
            sst[p, pl.ds(16 * k, 16)] = v & 0xFFFF
            dstg[p, pl.ds(16 * k, 16)] = v >> 16

    def issue_g(j, b):
        pltpu.async_copy(y_hbm.at[sst.at[b]], rows[b], gsem[b])

    def wait_g(j, b):
        pltpu.make_async_copy(
            y_hbm.at[sst.at[b]], rows[b], gsem[b]).wait()

    unpack(0, 0)
    unpack(1, 1)
    issue_g(0, 0)

    def body(i, carry):
        for b in range(2):
            j = 2 * i + b
            nb = 1 - b
            wait_g(j, b)
            if b == 0:
                issue_g(j + 1, nb)
            else:
                @pl.when(i <= CHT // 2 - 2)
                def _():
                    issue_g(j + 1, nb)
            pltpu.sync_copy(rows[b], acc_sh.at[dstg.at[b]], add=True)

            @pl.when(i <= CHT // 2 - 2)
            def _():
                unpack(j + 2, b)
        return carry

    lax.fori_loop(0, CHT // 2, body, 0)
    plsc.subcore_barrier()
    pltpu.sync_copy(acc_sh.at[pl.ds(s * RPT, RPT)],
                    out_hbm.at[c, pl.ds(s * RPT, RPT)])


def _tcA_body(x_ref, w_ref, degp_ref, y_ref, dinv_ref):
    deg = jnp.sum(degp_ref[...], axis=0) + 1.0
    dinv = lax.rsqrt(deg).reshape(BR, 1)
    xw = jnp.dot(x_ref[...], w_ref[...], preferred_element_type=jnp.float32)
    y_ref[...] = xw * dinv
    dinv_ref[...] = dinv


def _tcB_body(y1_ref, p_ref, dinv_ref, b1_ref, g_ref, be_ref, w2_ref,
              y2_ref):
    dinv = dinv_ref[...]
    h = (p_ref[0] + p_ref[1] + y1_ref[...]) * dinv + b1_ref[...]
    h = h * (g_ref[...] * (1.0 / jnp.sqrt(1.0 + 1e-5))) + be_ref[...]
    h = jnp.maximum(h, 0.0)
    y2_ref[...] = jnp.dot(
        h, w2_ref[...], preferred_element_type=jnp.float32) * dinv


def _tcC_body(y2_ref, p_ref, dinv_ref, b2_ref, o_ref):
    o = (p_ref[0] + p_ref[1] + y2_ref[...]) * dinv_ref[...] + b2_ref[...]
    m = jnp.max(o, axis=1, keepdims=True)
    e = jnp.exp(o - m)
    o_ref[...] = o - m - jnp.log(jnp.sum(e, axis=1, keepdims=True))


_row_spec = pl.BlockSpec((BR, D), lambda i: (i, 0))
_vec_spec = pl.BlockSpec((1, D), lambda i: (0, 0))
_w_spec = pl.BlockSpec((D, D), lambda i: (0, 0))
_dinv_spec = pl.BlockSpec((BR, 1), lambda i: (i, 0))
_p_spec = pl.BlockSpec((NC, BR, D), lambda i: (0, i, 0))

_tcA = pl.pallas_call(
    _tcA_body,
    grid=(GRID,),
    in_specs=[_row_spec, _w_spec, pl.BlockSpec((NW, BR), lambda i: (0, i))],
    out_specs=[_row_spec, _dinv_spec],
    out_shape=[jax.ShapeDtypeStruct((NPAD, D), jnp.float32),
               jax.ShapeDtypeStruct((NPAD, 1), jnp.float32)],
)

_tcB = pl.pallas_call(
    _tcB_body,
    grid=(GRID,),
    in_specs=[_row_spec, _p_spec, _dinv_spec, _vec_spec, _vec_spec,
              _vec_spec, _w_spec],
    out_specs=_row_spec,
    out_shape=jax.ShapeDtypeStruct((NPAD, D), jnp.float32),
)

_tcC = pl.pallas_call(
    _tcC_body,
    grid=(GRID,),
    in_specs=[_row_spec, _p_spec, _dinv_spec, _vec_spec],
    out_specs=_row_spec,
    out_shape=jax.ShapeDtypeStruct((NPAD, D), jnp.float32),
)


def kernel(x, edge_index, W1, b1, gamma, beta, W2, b2):
    src = edge_index[0]
    dst = edge_index[1]
    pad_e = E_PAD - E
    src_p = jnp.concatenate([src, jnp.zeros((pad_e,), jnp.int32)])
    dst_p = jnp.concatenate([dst, jnp.full((pad_e,), N, jnp.int32)])
    pk3d = (src_p | (dst_p << 16)).reshape(NW, CHT, 128)
    x_pad = jnp.pad(x, ((0, NPAD - N), (0, 0)))
    zero = jnp.zeros((NPAD, D), jnp.float32)

    degp = _deg_kernel(dst_p)
    y1, dinv = _tcA(x_pad, W1, degp)
    p1 = _scatter_kernel(y1, pk3d, zero)
    y2 = _tcB(y1, p1, dinv, b1.reshape(1, D), gamma.reshape(1, D),
              beta.reshape(1, D), W2)
    p2 = _scatter_kernel(y2, pk3d, zero)
    out = _tcC(y2, p2, dinv, b2.reshape(1, D))
    return out[:N]


# speedup vs baseline: 1.0668x; 1.0668x over previous
import functools

import jax
import jax.numpy as jnp
from jax import lax
from jax.experimental import pallas as pl
from jax.experimental.pallas import tpu as pltpu
from jax.experimental.pallas import tpu_sc as plsc

N = 10000
D = 128
E = 320000

NC = 2
NS = 16
NW = NC * NS

NPAD = 10240
RPT = NPAD // NS
CHT = 40
CHW = 256
EPT = CHT * CHW
E_PAD = EPT * NW

BR = 512
GRID = NPAD // BR

_MESH = plsc.VectorSubcoreMesh(
    core_axis_name="c", subcore_axis_name="s", num_cores=NC, num_subcores=NS)


@functools.partial(
    pl.kernel,
    out_type=jax.ShapeDtypeStruct((NW, NPAD), jnp.float32),
    mesh=_MESH,
    compiler_params=pltpu.CompilerParams(needs_layout_passes=False),
    scratch_types=[
        pltpu.VMEM((EPT,), jnp.int32),
        pltpu.VMEM((NPAD,), jnp.float32),
    ],
)
def _deg_kernel(dst_hbm, out_hbm, dst_v, acc_v):
    c = lax.axis_index("c")
    s = lax.axis_index("s")
    w = s * NC + c
    pltpu.sync_copy(dst_hbm.at[pl.ds(w * EPT, EPT)], dst_v)
    zeros = jnp.zeros((16,), jnp.float32)

    def zbody(i, carry):
        acc_v[pl.ds(i * 16, 16)] = zeros
        return carry

    lax.fori_loop(0, NPAD // 16, zbody, 0)
    ones = jnp.ones((16,), jnp.float32)

    def body(i, carry):
        idx = dst_v[pl.ds(i * 16, 16)]
        plsc.addupdate_scatter(acc_v, [idx], ones)
        return carry

    lax.fori_loop(0, EPT // 16, body, 0)
    pltpu.sync_copy(acc_v, out_hbm.at[w])


@functools.partial(
    pl.kernel,
    out_type=jax.ShapeDtypeStruct((NC, NPAD, D), jnp.float32),
    mesh=_MESH,
    scratch_types=[
        pltpu.VMEM((CHT * 2, 128), jnp.int32),
        pltpu.VMEM((2, 1, CHW), jnp.int32),
        pltpu.VMEM((2, 1, CHW), jnp.int32),
        pltpu.VMEM((CHW, D), jnp.float32),
        pltpu.VMEM_SHARED((NPAD, D), jnp.float32),
        pltpu.SemaphoreType.DMA,
    ],
)
def _scatter_kernel(y_hbm, pk_hbm, zero_hbm, out_hbm,
                    pk_v, sst, dstg, rows_v, acc_sh, gsem):
    c = lax.axis_index("c")
    s = lax.axis_index("s")
    w = s * NC + c

    @pl.when(c == 0)
    def _():
        pltpu.sync_copy(y_hbm.at[pl.ds(s * RPT, RPT)],
                        acc_sh.at[pl.ds(s * RPT, RPT)])

    @pl.when(c != 0)
    def _():
        pltpu.sync_copy(zero_hbm.at[pl.ds(s * RPT, RPT)],
                        acc_sh.at[pl.ds(s * RPT, RPT)])

    pltpu.sync_copy(pk_hbm.at[w], pk_v)
    plsc.subcore_barrier()

    def unpack(j, p):
        for r in range(2):
            for k in range(8):
                v = pk_v[2 * j + r, pl.ds(16 * k, 16)]
                sst[p, 0, pl.ds(128 * r + 16 * k, 16)] = v & 0xFFFF
                dstg[p, 0, pl.ds(128 * r + 16 * k, 16)] = v >> 16

    unpack(0, 0)

    def body(j, carry):
        p = j % 2
        pltpu.async_copy(y_hbm.at[sst.at[p, 0]], rows_v, gsem).wait()

        @pl.when(j <= CHT - 2)
        def _():
            unpack(j + 1, 1 - p)
        pltpu.sync_copy(rows_v, acc_sh.at[dstg.at[p, 0]], add=True)
        return carry

    lax.fori_loop(0, CHT, body, 0)
    plsc.subcore_barrier()
    pltpu.sync_copy(acc_sh.at[pl.ds(s * RPT, RPT)],
                    out_hbm.at[c, pl.ds(s * RPT, RPT)])


def _tcA_body(x_ref, w_ref, degp_ref, y_ref, dinv_ref):
    deg = jnp.sum(degp_ref[...], axis=0) + 1.0
    dinv = lax.rsqrt(deg).reshape(BR, 1)
    xw = jnp.dot(x_ref[...], w_ref[...], preferred_element_type=jnp.float32)
    y_ref[...] = xw * dinv
    dinv_ref[...] = dinv


def _tcB_body(p_ref, dinv_ref, b1_ref, g_ref, be_ref, w2_ref, y2_ref):
    dinv = dinv_ref[...]
    h = (p_ref[0] + p_ref[1]) * dinv + b1_ref[...]
    h = h * (g_ref[...] * (1.0 / jnp.sqrt(1.0 + 1e-5))) + be_ref[...]
    h = jnp.maximum(h, 0.0)
    y2_ref[...] = jnp.dot(
        h, w2_ref[...], preferred_element_type=jnp.float32) * dinv


def _tcC_body(p_ref, dinv_ref, b2_ref, o_ref):
    o = (p_ref[0] + p_ref[1]) * dinv_ref[...] + b2_ref[...]
    m = jnp.max(o, axis=1, keepdims=True)
    e = jnp.exp(o - m)
    o_ref[...] = o - m - jnp.log(jnp.sum(e, axis=1, keepdims=True))


_row_spec = pl.BlockSpec((BR, D), lambda i: (i, 0))
_vec_spec = pl.BlockSpec((1, D), lambda i: (0, 0))
_w_spec = pl.BlockSpec((D, D), lambda i: (0, 0))
_dinv_spec = pl.BlockSpec((BR, 1), lambda i: (i, 0))
_p_spec = pl.BlockSpec((NC, BR, D), lambda i: (0, i, 0))

_tcA = pl.pallas_call(
    _tcA_body,
    grid=(GRID,),
    in_specs=[_row_spec, _w_spec, pl.BlockSpec((NW, BR), lambda i: (0, i))],
    out_specs=[_row_spec, _dinv_spec],
    out_shape=[jax.ShapeDtypeStruct((NPAD, D), jnp.float32),
               jax.ShapeDtypeStruct((NPAD, 1), jnp.float32)],
)

_tcB = pl.pallas_call(
    _tcB_body,
    grid=(GRID,),
    in_specs=[_p_spec, _dinv_spec, _vec_spec, _vec_spec, _vec_spec,
              _w_spec],
    out_specs=_row_spec,
    out_shape=jax.ShapeDtypeStruct((NPAD, D), jnp.float32),
)

_tcC = pl.pallas_call(
    _tcC_body,
    grid=(GRID,),
    in_specs=[_p_spec, _dinv_spec, _vec_spec],
    out_specs=_row_spec,
    out_shape=jax.ShapeDtypeStruct((NPAD, D), jnp.float32),
)


def kernel(x, edge_index, W1, b1, gamma, beta, W2, b2):
    src = edge_index[0]
    dst = edge_index[1]
    pad_e = E_PAD - E
    src_p = jnp.concatenate([src, jnp.zeros((pad_e,), jnp.int32)])
    dst_p = jnp.concatenate([dst, jnp.full((pad_e,), N, jnp.int32)])
    pk3d = (src_p | (dst_p << 16)).reshape(NW, CHT * 2, 128)
    x_pad = jnp.pad(x, ((0, NPAD - N), (0, 0)))
    zero = jnp.zeros((NPAD, D), jnp.float32)

    degp = _deg_kernel(dst_p)
    y1, dinv = _tcA(x_pad, W1, degp)
    p1 = _scatter_kernel(y1, pk3d, zero)
    y2 = _tcB(p1, dinv, b1.reshape(1, D), gamma.reshape(1, D),
              beta.reshape(1, D), W2)
    p2 = _scatter_kernel(y2, pk3d, zero)
    out = _tcC(p2, dinv, b2.reshape(1, D))
    return out[:N]


# speedup vs baseline: 1.6181x; 1.5168x over previous
import functools

import jax
import jax.numpy as jnp
from jax import lax
from jax.experimental import pallas as pl
from jax.experimental.pallas import tpu as pltpu
from jax.experimental.pallas import tpu_sc as plsc

N = 10000
D = 128
E = 320000

NC = 2
NS = 16
NW = NC * NS

NPAD = 10240
RPT = NPAD // NS
CHT = 79
EPT = CHT * 128
E_PAD = EPT * NW

BR = 512
GRID = NPAD // BR

_MESH = plsc.VectorSubcoreMesh(
    core_axis_name="c", subcore_axis_name="s", num_cores=NC, num_subcores=NS)


@functools.partial(
    pl.kernel,
    out_type=jax.ShapeDtypeStruct((NW, NPAD), jnp.float32),
    mesh=_MESH,
    compiler_params=pltpu.CompilerParams(needs_layout_passes=False),
    scratch_types=[
        pltpu.VMEM((EPT,), jnp.int32),
        pltpu.VMEM((NPAD,), jnp.float32),
    ],
)
def _deg_kernel(dst_hbm, out_hbm, dst_v, acc_v):
    c = lax.axis_index("c")
    s = lax.axis_index("s")
    w = s * NC + c
    pltpu.sync_copy(dst_hbm.at[pl.ds(w * EPT, EPT)], dst_v)
    zeros = jnp.zeros((16,), jnp.float32)

    def zbody(i, carry):
        acc_v[pl.ds(i * 16, 16)] = zeros
        return carry

    lax.fori_loop(0, NPAD // 16, zbody, 0)
    ones = jnp.ones((16,), jnp.float32)

    def body(i, carry):
        idx = dst_v[pl.ds(i * 16, 16)]
        plsc.addupdate_scatter(acc_v, [idx], ones)
        return carry

    lax.fori_loop(0, EPT // 16, body, 0)
    pltpu.sync_copy(acc_v, out_hbm.at[w])


@functools.partial(
    pl.kernel,
    out_type=jax.ShapeDtypeStruct((NC, NPAD, D), jnp.float32),
    mesh=_MESH,
    scratch_types=[
        pltpu.VMEM((CHT, 128), jnp.int32),
        pltpu.VMEM((CHT, 128), jnp.int32),
        pltpu.VMEM((128, D), jnp.float32),
        pltpu.VMEM_SHARED((NPAD, D), jnp.float32),
        pltpu.SemaphoreType.DMA,
    ],
)
def _scatter_kernel(y_hbm, src_hbm, dst_hbm, zero_hbm, out_hbm,
                    src_v, dst_v, rows_v, acc_sh, sem):
    c = lax.axis_index("c")
    s = lax.axis_index("s")
    w = s * NC + c
    pltpu.sync_copy(zero_hbm.at[pl.ds(s * RPT, RPT)],
                    acc_sh.at[pl.ds(s * RPT, RPT)])
    pltpu.sync_copy(src_hbm.at[w], src_v)
    pltpu.sync_copy(dst_hbm.at[w], dst_v)
    plsc.subcore_barrier()

    def body(j, carry):
        pltpu.async_copy(y_hbm.at[src_v.at[j]], rows_v, sem).wait()
        pltpu.sync_copy(rows_v, acc_sh.at[dst_v.at[j]], add=True)
        return carry

    lax.fori_loop(0, CHT, body, 0)
    plsc.subcore_barrier()
    pltpu.sync_copy(acc_sh.at[pl.ds(s * RPT, RPT)],
                    out_hbm.at[c, pl.ds(s * RPT, RPT)])


def _tcA_body(x_ref, w_ref, degp_ref, y_ref, dinv_ref):
    deg = jnp.sum(degp_ref[...], axis=0) + 1.0
    dinv = lax.rsqrt(deg).reshape(BR, 1)
    xw = jnp.dot(x_ref[...], w_ref[...], preferred_element_type=jnp.float32)
    y_ref[...] = xw * dinv
    dinv_ref[...] = dinv


def _tcB_body(y1_ref, p_ref, dinv_ref, b1_ref, g_ref, be_ref, w2_ref, y2_ref):
    dinv = dinv_ref[...]
    h = (p_ref[0] + p_ref[1] + y1_ref[...]) * dinv + b1_ref[...]
    h = h * (g_ref[...] * (1.0 / jnp.sqrt(1.0 + 1e-5))) + be_ref[...]
    h = jnp.maximum(h, 0.0)
    y2_ref[...] = jnp.dot(
        h, w2_ref[...], preferred_element_type=jnp.float32) * dinv


def _tcC_body(y2_ref, p_ref, dinv_ref, b2_ref, o_ref):
    o = (p_ref[0] + p_ref[1] + y2_ref[...]) * dinv_ref[...] + b2_ref[...]
    m = jnp.max(o, axis=1, keepdims=True)
    e = jnp.exp(o - m)
    o_ref[...] = o - m - jnp.log(jnp.sum(e, axis=1, keepdims=True))


_row_spec = pl.BlockSpec((BR, D), lambda i: (i, 0))
_vec_spec = pl.BlockSpec((1, D), lambda i: (0, 0))
_w_spec = pl.BlockSpec((D, D), lambda i: (0, 0))
_dinv_spec = pl.BlockSpec((BR, 1), lambda i: (i, 0))
_p_spec = pl.BlockSpec((NC, BR, D), lambda i: (0, i, 0))

_tcA = pl.pallas_call(
    _tcA_body,
    grid=(GRID,),
    in_specs=[_row_spec, _w_spec, pl.BlockSpec((NW, BR), lambda i: (0, i))],
    out_specs=[_row_spec, _dinv_spec],
    out_shape=[jax.ShapeDtypeStruct((NPAD, D), jnp.float32),
               jax.ShapeDtypeStruct((NPAD, 1), jnp.float32)],
)

_tcB = pl.pallas_call(
    _tcB_body,
    grid=(GRID,),
    in_specs=[_row_spec, _p_spec, _dinv_spec, _vec_spec, _vec_spec, _vec_spec,
              _w_spec],
    out_specs=_row_spec,
    out_shape=jax.ShapeDtypeStruct((NPAD, D), jnp.float32),
)

_tcC = pl.pallas_call(
    _tcC_body,
    grid=(GRID,),
    in_specs=[_row_spec, _p_spec, _dinv_spec, _vec_spec],
    out_specs=_row_spec,
    out_shape=jax.ShapeDtypeStruct((NPAD, D), jnp.float32),
)


def kernel(x, edge_index, W1, b1, gamma, beta, W2, b2):
    src = edge_index[0]
    dst = edge_index[1]
    pad_e = E_PAD - E
    src_p = jnp.concatenate([src, jnp.zeros((pad_e,), jnp.int32)])
    dst_p = jnp.concatenate([dst, jnp.full((pad_e,), N, jnp.int32)])
    src3d = src_p.reshape(NW, CHT, 128)
    dst3d = dst_p.reshape(NW, CHT, 128)
    x_pad = jnp.pad(x, ((0, NPAD - N), (0, 0)))
    zero = jnp.zeros((NPAD, D), jnp.float32)

    degp = _deg_kernel(dst_p)
    y1, dinv = _tcA(x_pad, W1, degp)
    p1 = _scatter_kernel(y1, src3d, dst3d, zero)
    y2 = _tcB(y1, p1, dinv, b1.reshape(1, D), gamma.reshape(1, D),
              beta.reshape(1, D), W2)
    p2 = _scatter_kernel(y2, src3d, dst3d, zero)
    out = _tcC(y2, p2, dinv, b2.reshape(1, D))
    return out[:N]


# speedup vs baseline: 1.7251x; 1.0661x over previous
import functools

import jax
import jax.numpy as jnp
from jax import lax
from jax.experimental import pallas as pl
from jax.experimental.pallas import tpu as pltpu
from jax.experimental.pallas import tpu_sc as plsc

N = 10000
D = 128
E = 320000

NC = 2
NS = 16
NW = NC * NS

NPAD = 10240
RPT = NPAD // NS
CHT = 79
EPT = CHT * 128
E_PAD = EPT * NW

BR = 512
GRID = NPAD // BR

_MESH = plsc.VectorSubcoreMesh(
    core_axis_name="c", subcore_axis_name="s", num_cores=NC, num_subcores=NS)


@functools.partial(
    pl.kernel,
    out_type=jax.ShapeDtypeStruct((NW, NPAD), jnp.float32),
    mesh=_MESH,
    compiler_params=pltpu.CompilerParams(needs_layout_passes=False),
    scratch_types=[
        pltpu.VMEM((EPT,), jnp.int32),
        pltpu.VMEM((NPAD,), jnp.float32),
    ],
)
def _deg_kernel(dst_hbm, out_hbm, dst_v, acc_v):
    c = lax.axis_index("c")
    s = lax.axis_index("s")
    w = s * NC + c
    pltpu.sync_copy(dst_hbm.at[pl.ds(w * EPT, EPT)], dst_v)
    zeros = jnp.zeros((16,), jnp.float32)

    def zbody(i, carry):
        acc_v[pl.ds(i * 16, 16)] = zeros
        return carry

    lax.fori_loop(0, NPAD // 16, zbody, 0)
    ones = jnp.ones((16,), jnp.float32)

    def body(i, carry):
        idx = dst_v[pl.ds(i * 16, 16)]
        plsc.addupdate_scatter(acc_v, [idx], ones)
        return carry

    lax.fori_loop(0, EPT // 16, body, 0)
    pltpu.sync_copy(acc_v, out_hbm.at[w])


@functools.partial(
    pl.kernel,
    out_type=jax.ShapeDtypeStruct((NC, NPAD, D), jnp.float32),
    mesh=_MESH,
    scratch_types=[
        pltpu.VMEM((CHT, 128), jnp.int32),
        pltpu.VMEM((CHT, 128), jnp.int32),
        pltpu.VMEM((128, D), jnp.float32),
        pltpu.VMEM_SHARED((NPAD, D), jnp.float32),
        pltpu.SemaphoreType.DMA,
    ],
)
def _scatter_kernel(y_hbm, src_hbm, dst_hbm, zero_hbm, out_hbm,
                    src_v, dst_v, rows_v, acc_sh, sem):
    c = lax.axis_index("c")
    s = lax.axis_index("s")
    w = s * NC + c
    @pl.when(c == 0)
    def _():
        pltpu.sync_copy(y_hbm.at[pl.ds(s * RPT, RPT)],
                        acc_sh.at[pl.ds(s * RPT, RPT)])

    @pl.when(c != 0)
    def _():
        pltpu.sync_copy(zero_hbm.at[pl.ds(s * RPT, RPT)],
                        acc_sh.at[pl.ds(s * RPT, RPT)])

    pltpu.sync_copy(src_hbm.at[w], src_v)
    pltpu.sync_copy(dst_hbm.at[w], dst_v)
    plsc.subcore_barrier()

    def body(j, carry):
        pltpu.async_copy(y_hbm.at[src_v.at[j]], rows_v, sem).wait()
        pltpu.sync_copy(rows_v, acc_sh.at[dst_v.at[j]], add=True)
        return carry

    lax.fori_loop(0, CHT, body, 0)
    plsc.subcore_barrier()
    pltpu.sync_copy(acc_sh.at[pl.ds(s * RPT, RPT)],
                    out_hbm.at[c, pl.ds(s * RPT, RPT)])


def _tcA_body(x_ref, w_ref, degp_ref, y_ref, dinv_ref):
    deg = jnp.sum(degp_ref[...], axis=0) + 1.0
    dinv = lax.rsqrt(deg).reshape(BR, 1)
    xw = jnp.dot(x_ref[...], w_ref[...], preferred_element_type=jnp.float32)
    y_ref[...] = xw * dinv
    dinv_ref[...] = dinv


def _tcB_body(p_ref, dinv_ref, b1_ref, g_ref, be_ref, w2_ref, y2_ref):
    dinv = dinv_ref[...]
    h = (p_ref[0] + p_ref[1]) * dinv + b1_ref[...]
    h = h * (g_ref[...] * (1.0 / jnp.sqrt(1.0 + 1e-5))) + be_ref[...]
    h = jnp.maximum(h, 0.0)
    y2_ref[...] = jnp.dot(
        h, w2_ref[...], preferred_element_type=jnp.float32) * dinv


def _tcC_body(p_ref, dinv_ref, b2_ref, o_ref):
    o = (p_ref[0] + p_ref[1]) * dinv_ref[...] + b2_ref[...]
    m = jnp.max(o, axis=1, keepdims=True)
    e = jnp.exp(o - m)
    o_ref[...] = o - m - jnp.log(jnp.sum(e, axis=1, keepdims=True))


_row_spec = pl.BlockSpec((BR, D), lambda i: (i, 0))
_vec_spec = pl.BlockSpec((1, D), lambda i: (0, 0))
_w_spec = pl.BlockSpec((D, D), lambda i: (0, 0))
_dinv_spec = pl.BlockSpec((BR, 1), lambda i: (i, 0))
_p_spec = pl.BlockSpec((NC, BR, D), lambda i: (0, i, 0))

_tcA = pl.pallas_call(
    _tcA_body,
    grid=(GRID,),
    in_specs=[_row_spec, _w_spec, pl.BlockSpec((NW, BR), lambda i: (0, i))],
    out_specs=[_row_spec, _dinv_spec],
    out_shape=[jax.ShapeDtypeStruct((NPAD, D), jnp.float32),
               jax.ShapeDtypeStruct((NPAD, 1), jnp.float32)],
)

_tcB = pl.pallas_call(
    _tcB_body,
    grid=(GRID,),
    in_specs=[_p_spec, _dinv_spec, _vec_spec, _vec_spec, _vec_spec,
              _w_spec],
    out_specs=_row_spec,
    out_shape=jax.ShapeDtypeStruct((NPAD, D), jnp.float32),
)

_tcC = pl.pallas_call(
    _tcC_body,
    grid=(GRID,),
    in_specs=[_p_spec, _dinv_spec, _vec_spec],
    out_specs=_row_spec,
    out_shape=jax.ShapeDtypeStruct((NPAD, D), jnp.float32),
)


def kernel(x, edge_index, W1, b1, gamma, beta, W2, b2):
    src = edge_index[0]
    dst = edge_index[1]
    pad_e = E_PAD - E
    src_p = jnp.concatenate([src, jnp.zeros((pad_e,), jnp.int32)])
    dst_p = jnp.concatenate([dst, jnp.full((pad_e,), N, jnp.int32)])
    src3d = src_p.reshape(NW, CHT, 128)
    dst3d = dst_p.reshape(NW, CHT, 128)
    x_pad = jnp.pad(x, ((0, NPAD - N), (0, 0)))
    zero = jnp.zeros((NPAD, D), jnp.float32)

    degp = _deg_kernel(dst_p)
    y1, dinv = _tcA(x_pad, W1, degp)
    p1 = _scatter_kernel(y1, src3d, dst3d, zero)
    y2 = _tcB(p1, dinv, b1.reshape(1, D), gamma.reshape(1, D),
              beta.reshape(1, D), W2)
    p2 = _scatter_kernel(y2, src3d, dst3d, zero)
    out = _tcC(p2, dinv, b2.reshape(1, D))
    return out[:N]
